# baseline (speedup 1.0000x reference)
import jax
import jax.numpy as jnp
from jax.experimental import pallas as pl
from jax.experimental.pallas import tpu as pltpu


def _pair_kernel(x_ref, w1cat_ref, b1_ref, w2_ref, b2_ref, out_ref,
                 at_scr, bv_scr, w2bc_scr, s_scr):
    t = pl.program_id(1)
    T = pl.num_programs(1)
    H = w2_ref.shape[1]
    n = at_scr.shape[1]
    R = n // T

    K = at_scr.shape[0] // H

    @pl.when(t == 0)
    def _init():
        tx = jnp.tanh(x_ref[0])
        ab = jnp.dot(tx, w1cat_ref[:], preferred_element_type=jnp.float32)
        atb = (ab[:, :H] + b1_ref[:]).T.astype(jnp.bfloat16)
        w2b = jnp.broadcast_to(w2_ref[:].T, (H, n)).astype(jnp.bfloat16)
        at_scr[:] = jnp.concatenate([atb] * K, axis=0)
        w2bc_scr[:] = jnp.concatenate([w2b] * K, axis=0)
        bv_scr[:] = ab[:, H:].astype(jnp.bfloat16).reshape(n // K, K * H)

    base = t * R
    at = at_scr[:]
    w2bc = w2bc_scr[:]
    b2v = b2_ref[0, 0]
    for r0 in range(0, R, K):
        bcol = bv_scr[r0 // K][:, None]
        p = jnp.tanh(at + bcol) * w2bc
        rows = []
        for k in range(K):
            q = p[k * H:(k + 1) * H]
            h = H
            while h > 16:
                h //= 2
                q = q[:h] + q[h:]
            rows.append(jnp.sum(q, axis=0, dtype=jnp.float32))
        s_k = jnp.stack(rows, axis=0)
        s_scr[r0:r0 + K, :] = s_k + b2v

    @pl.when(t == T - 1)
    def _finalize():
        sv = s_scr[:]
        out_ref[0] = sv + sv.T


def kernel(x, W1, b1, W2, b2):
    B, n, F = x.shape
    H = W1.shape[0]
    T = 1

    w1cat = jnp.concatenate([W1[:, :F].T, W1[:, F:].T], axis=1)
    b1r = b1.reshape(1, H)
    w2r = W2.reshape(1, H)
    b2r = b2.reshape(1, 1)

    return pl.pallas_call(
        _pair_kernel,
        grid=(B, T),
        in_specs=[
            pl.BlockSpec((1, n, F), lambda b, t: (b, 0, 0)),
            pl.BlockSpec((F, 2 * H), lambda b, t: (0, 0)),
            pl.BlockSpec((1, H), lambda b, t: (0, 0)),
            pl.BlockSpec((1, H), lambda b, t: (0, 0)),
            pl.BlockSpec((1, 1), lambda b, t: (0, 0)),
        ],
        out_specs=pl.BlockSpec((1, n, n), lambda b, t: (b, 0, 0)),
        out_shape=jax.ShapeDtypeStruct((B, n, n), jnp.float32),
        scratch_shapes=[
            pltpu.VMEM((4 * H, n), jnp.bfloat16),
            pltpu.VMEM((n // 4, 4 * H), jnp.bfloat16),
            pltpu.VMEM((4 * H, n), jnp.bfloat16),
            pltpu.VMEM((n, n), jnp.float32),
        ],
        compiler_params=pltpu.CompilerParams(
            dimension_semantics=("parallel", "arbitrary"),
        ),
    )(x, w1cat, b1r, w2r, b2r)


# speedup vs baseline: 1.0340x; 1.0340x over previous
import jax
import jax.numpy as jnp
from jax.experimental import pallas as pl
from jax.experimental.pallas import tpu as pltpu


def _pair_kernel(x_ref, w1cat_ref, b1_ref, w2_ref, b2_ref, out_ref,
                 at_scr, bv_scr, w2bc_scr, s_scr):
    t = pl.program_id(1)
    T = pl.num_programs(1)
    H = w2_ref.shape[1]
    n = at_scr.shape[1]
    R = n // T

    K = at_scr.shape[0] // H

    @pl.when(t == 0)
    def _init():
        tx = jnp.tanh(x_ref[0])
        ab = jnp.dot(tx, w1cat_ref[:], preferred_element_type=jnp.float32)
        atb = (ab[:, :H] + b1_ref[:]).T.astype(jnp.bfloat16)
        w2b = jnp.broadcast_to(w2_ref[:].T, (H, n)).astype(jnp.bfloat16)
        at_scr[:] = jnp.concatenate([atb] * K, axis=0)
        w2bc_scr[:] = jnp.concatenate([w2b] * K, axis=0)
        bv_scr[:] = ab[:, H:].astype(jnp.bfloat16).reshape(n // K, K * H)

    base = t * R
    at = at_scr[:]
    w2bc = w2bc_scr[:]
    b2v = b2_ref[0, 0]
    w2row = w2_ref[:].astype(jnp.bfloat16)
    for r0 in range(0, R, K):
        bcol = bv_scr[r0 // K][:, None]
        z = jnp.tanh(at + bcol)
        rows = []
        for k in range(K // 2):
            q = z[k * H:(k + 1) * H] * w2bc[k * H:(k + 1) * H]
            h = H
            while h > 16:
                h //= 2
                q = q[:h] + q[h:]
            rows.append(jnp.sum(q, axis=0, dtype=jnp.float32))
        s_k = jnp.stack(rows, axis=0)
        s_scr[r0:r0 + K // 2, :] = s_k + b2v
        for k in range(K // 2, K):
            s_r = jnp.dot(w2row, z[k * H:(k + 1) * H],
                          preferred_element_type=jnp.float32)
            s_scr[r0 + k:r0 + k + 1, :] = s_r + b2v

    @pl.when(t == T - 1)
    def _finalize():
        sv = s_scr[:]
        out_ref[0] = sv + sv.T


def kernel(x, W1, b1, W2, b2):
    B, n, F = x.shape
    H = W1.shape[0]
    T = 1

    w1cat = jnp.concatenate([W1[:, :F].T, W1[:, F:].T], axis=1)
    b1r = b1.reshape(1, H)
    w2r = W2.reshape(1, H)
    b2r = b2.reshape(1, 1)

    return pl.pallas_call(
        _pair_kernel,
        grid=(B, T),
        in_specs=[
            pl.BlockSpec((1, n, F), lambda b, t: (b, 0, 0)),
            pl.BlockSpec((F, 2 * H), lambda b, t: (0, 0)),
            pl.BlockSpec((1, H), lambda b, t: (0, 0)),
            pl.BlockSpec((1, H), lambda b, t: (0, 0)),
            pl.BlockSpec((1, 1), lambda b, t: (0, 0)),
        ],
        out_specs=pl.BlockSpec((1, n, n), lambda b, t: (b, 0, 0)),
        out_shape=jax.ShapeDtypeStruct((B, n, n), jnp.float32),
        scratch_shapes=[
            pltpu.VMEM((4 * H, n), jnp.bfloat16),
            pltpu.VMEM((n // 4, 4 * H), jnp.bfloat16),
            pltpu.VMEM((4 * H, n), jnp.bfloat16),
            pltpu.VMEM((n, n), jnp.float32),
        ],
        compiler_params=pltpu.CompilerParams(
            dimension_semantics=("parallel", "arbitrary"),
        ),
    )(x, w1cat, b1r, w2r, b2r)
